# baseline (speedup 1.0000x reference)
import functools
import math

import jax
import jax.numpy as jnp
from jax import lax
from jax.experimental import pallas as pl
from jax.experimental.pallas import tpu as pltpu
from jax.experimental.pallas import tpu_sc as plsc

D_MODEL = 64
SCALE = math.sqrt(D_MODEL)
LANES = 16
IDX_SPLIT = 128
IDX_BLOCK = 32


def _row_groups(seq_len):
    starts = list(range(0, seq_len - LANES + 1, LANES))
    if seq_len % LANES != 0:
        starts.append(seq_len - LANES)
    return starts


def _make_kernel(n_seq, seq_len):
    info = plsc.get_sparse_core_info()
    nc, ns = info.num_cores, info.num_subcores
    nw = nc * ns
    assert n_seq % nw == 0
    seq_per_w = n_seq // nw
    assert seq_per_w % IDX_BLOCK == 0
    n_blocks = seq_per_w // IDX_BLOCK
    n_pairs = IDX_BLOCK // 2
    rem = seq_len - IDX_SPLIT
    groups = _row_groups(seq_len)

    mesh = plsc.VectorSubcoreMesh(core_axis_name="c", subcore_axis_name="s")

    @functools.partial(
        pl.kernel,
        mesh=mesh,
        out_type=jax.ShapeDtypeStruct((n_seq, seq_len, D_MODEL), jnp.float32),
        scratch_types=[
            pltpu.VMEM((IDX_BLOCK, seq_len), jnp.int32),
            pltpu.VMEM((IDX_BLOCK, seq_len), jnp.int32),
            pltpu.VMEM((seq_len, 2 * D_MODEL), jnp.float32),
            pltpu.VMEM((seq_len, 2 * D_MODEL), jnp.float32),
            pltpu.VMEM((seq_len, D_MODEL), jnp.float32),
            pltpu.VMEM((seq_len, D_MODEL), jnp.float32),
            pltpu.SemaphoreType.DMA,
            pltpu.SemaphoreType.DMA,
            pltpu.SemaphoreType.DMA,
            pltpu.SemaphoreType.DMA,
        ],
        compiler_params=pltpu.CompilerParams(use_tc_tiling_on_sc=True, needs_layout_passes=False),
    )
    def body(x_hbm, table_hbm, out_hbm, idx_v, idx2_v, rows0, rows1,
             outb0, outb1, gsem0, gsem1, osem0, osem1):
        wid = lax.axis_index("s") * nc + lax.axis_index("c")
        base = wid * seq_per_w
        rows_v = (rows0, rows1)
        outb_v = (outb0, outb1)
        gsem = (gsem0, gsem1)
        osem = (osem0, osem1)
        iota = lax.iota(jnp.int32, LANES)

        def transform_indices():
            def tr_body(sl, _):
                for r0 in groups:
                    slc = pl.ds(r0, LANES)
                    idx2_v[sl, slc] = lax.shift_right_logical(
                        idx_v[sl, slc], 1)
                return 0

            lax.fori_loop(0, IDX_BLOCK, tr_body, 0)

        def start_gather(sl, b):
            pltpu.async_copy(
                table_hbm.at[idx2_v.at[sl, pl.ds(0, IDX_SPLIT)]],
                rows_v[b].at[pl.ds(0, IDX_SPLIT)], gsem[b])
            pltpu.async_copy(
                table_hbm.at[idx2_v.at[sl, pl.ds(IDX_SPLIT, rem)]],
                rows_v[b].at[pl.ds(IDX_SPLIT, rem)], gsem[b])

        def wait_gather(sl, b):
            pltpu.make_async_copy(
                table_hbm.at[idx2_v.at[sl, pl.ds(0, IDX_SPLIT)]],
                rows_v[b].at[pl.ds(0, IDX_SPLIT)], gsem[b]).wait()
            pltpu.make_async_copy(
                table_hbm.at[idx2_v.at[sl, pl.ds(IDX_SPLIT, rem)]],
                rows_v[b].at[pl.ds(IDX_SPLIT, rem)], gsem[b]).wait()

        def start_writeback(s, b):
            pltpu.async_copy(outb_v[b], out_hbm.at[base + s], osem[b])

        def wait_writeback(s, b):
            pltpu.make_async_copy(outb_v[b], out_hbm.at[base + s],
                                  osem[b]).wait()

        def select_scale(sl, b):
            rows = rows_v[b]
            outb = outb_v[b]
            for r0 in groups:
                iv = idx_v[sl, pl.ds(r0, LANES)]
                off = (iv & 1) * D_MODEL
                ridx = r0 + iota

                def d_body(dg, _, off=off, ridx=ridx):
                    for u in range(8):
                        d = dg * 8 + u
                        v = plsc.load_gather(rows, [ridx, off + d])
                        plsc.store_scatter(outb, [ridx, iota * 0 + d],
                                           v * SCALE)
                    return 0

                lax.fori_loop(0, D_MODEL // 8, d_body, 0)

        def blk_body(blk, _):
            blk_s = blk * IDX_BLOCK
            pltpu.sync_copy(x_hbm.at[pl.ds(base + blk_s, IDX_BLOCK)], idx_v)
            transform_indices()

            @pl.when(blk > 0)
            def _():
                wait_writeback(blk_s - 1, 1)

            start_gather(0, 0)

            def pair_body(p, _):
                l0 = 2 * p
                l1 = l0 + 1
                wait_gather(l0, 0)

                @pl.when(p > 0)
                def _():
                    wait_writeback(blk_s + l0 - 1, 1)

                start_gather(l1, 1)
                select_scale(l0, 0)
                start_writeback(blk_s + l0, 0)
                wait_gather(l1, 1)
                wait_writeback(blk_s + l0, 0)

                @pl.when(p < n_pairs - 1)
                def _():
                    start_gather(l0 + 2, 0)

                select_scale(l1, 1)
                start_writeback(blk_s + l1, 1)
                return 0

            lax.fori_loop(0, n_pairs, pair_body, 0)
            return 0

        lax.fori_loop(0, n_blocks, blk_body, 0)
        wait_writeback(seq_per_w - 1, 1)

    return body


def kernel(x, table):
    n_seq, seq_len = x.shape
    table_pairs = table.reshape(500000, 128)
    return _make_kernel(n_seq, seq_len)(x.astype(jnp.int32), table_pairs)


# speedup vs baseline: 2.4879x; 2.4879x over previous
import functools
import math

import jax
import jax.numpy as jnp
from jax import lax
from jax.experimental import pallas as pl
from jax.experimental.pallas import tpu as pltpu
from jax.experimental.pallas import tpu_sc as plsc

D_MODEL = 64
SCALE = math.sqrt(D_MODEL)
LANES = 16
SEQ_CHUNK = 4


def _make_kernel(n_seq, seq_len):
    info = plsc.get_sparse_core_info()
    nc, ns = info.num_cores, info.num_subcores
    nw = nc * ns
    assert n_seq % nw == 0
    seq_per_w = n_seq // nw
    assert seq_per_w % SEQ_CHUNK == 0
    n_chunks = seq_per_w // SEQ_CHUNK
    assert n_chunks % 2 == 0
    n_pairs = n_chunks // 2
    vregs_per_row = D_MODEL // LANES

    mesh = plsc.VectorSubcoreMesh(core_axis_name="c", subcore_axis_name="s")

    @functools.partial(
        pl.kernel,
        mesh=mesh,
        out_type=jax.ShapeDtypeStruct((n_seq, seq_len, D_MODEL), jnp.float32),
        scratch_types=[
            pltpu.VMEM((seq_per_w, seq_len), jnp.int32),
            pltpu.VMEM((SEQ_CHUNK, seq_len, D_MODEL), jnp.float32),
            pltpu.VMEM((SEQ_CHUNK, seq_len, D_MODEL), jnp.float32),
            pltpu.SemaphoreType.DMA,
            pltpu.SemaphoreType.DMA,
            pltpu.SemaphoreType.DMA,
            pltpu.SemaphoreType.DMA,
        ],
        compiler_params=pltpu.CompilerParams(use_tc_tiling_on_sc=False),
    )
    def body(x_hbm, table_hbm, out_hbm, idx_all, rows0, rows1,
             gsem0, gsem1, osem0, osem1):
        wid = lax.axis_index("s") * nc + lax.axis_index("c")
        base = wid * seq_per_w
        rows_v = (rows0, rows1)
        gsem = (gsem0, gsem1)
        osem = (osem0, osem1)

        pltpu.sync_copy(x_hbm.at[pl.ds(base, seq_per_w)], idx_all)

        def start_gather(c, b):
            for s in range(SEQ_CHUNK):
                pltpu.async_copy(
                    table_hbm.at[idx_all.at[c * SEQ_CHUNK + s]],
                    rows_v[b].at[s], gsem[b])

        def wait_gather(c, b):
            for s in range(SEQ_CHUNK):
                pltpu.make_async_copy(
                    table_hbm.at[idx_all.at[c * SEQ_CHUNK + s]],
                    rows_v[b].at[s], gsem[b]).wait()

        def start_writeback(c, b):
            pltpu.async_copy(
                rows_v[b],
                out_hbm.at[pl.ds(base + c * SEQ_CHUNK, SEQ_CHUNK)],
                osem[b])

        def wait_writeback(c, b):
            pltpu.make_async_copy(
                rows_v[b],
                out_hbm.at[pl.ds(base + c * SEQ_CHUNK, SEQ_CHUNK)],
                osem[b]).wait()

        def scale(b):
            rows = rows_v[b]

            def scale_body(r, _):
                for s in range(SEQ_CHUNK):
                    for k in range(vregs_per_row):
                        sl = pl.ds(k * LANES, LANES)
                        rows[s, r, sl] = rows[s, r, sl] * SCALE
                return 0

            lax.fori_loop(0, seq_len, scale_body, 0)

        start_gather(0, 0)

        def pair_body(p, _):
            c0 = 2 * p
            c1 = c0 + 1
            wait_gather(c0, 0)

            @pl.when(p > 0)
            def _():
                wait_writeback(c0 - 1, 1)

            start_gather(c1, 1)
            scale(0)
            start_writeback(c0, 0)
            wait_gather(c1, 1)
            wait_writeback(c0, 0)

            @pl.when(p < n_pairs - 1)
            def _():
                start_gather(c0 + 2, 0)

            scale(1)
            start_writeback(c1, 1)
            return 0

        lax.fori_loop(0, n_pairs, pair_body, 0)
        wait_writeback(n_chunks - 1, 1)

    return body


def kernel(x, table):
    n_seq, seq_len = x.shape
    return _make_kernel(n_seq, seq_len)(x.astype(jnp.int32), table)
